# baseline (speedup 1.0000x reference)
import jax
import jax.numpy as jnp
from jax.experimental import pallas as pl

_THRES = 0.5
_L = 4096
_BR = 256
_NB = _L // _BR


def _rowstat_body(x_ref, inds_ref, mask_ref):
    pi = pl.program_id(0)
    x = x_ref[...]
    col = jax.lax.broadcasted_iota(jnp.int32, (_BR, _L), 1)
    g = pi * _BR + jax.lax.broadcasted_iota(jnp.int32, (_BR, 1), 0)
    masked = jnp.where(col == g, -jnp.inf, x)
    vmax = jnp.max(masked, axis=1, keepdims=True)
    cand = jnp.where(masked == vmax, col, _L)
    inds = jnp.min(cand, axis=1, keepdims=True)
    m = (vmax > _THRES).astype(jnp.int32)
    inds_ref[...] = inds[None]
    mask_ref[...] = m[None]


def _assemble_body(inds_c_ref, inds_r_ref, mask_c_ref, mask_r_ref, out_ref):
    pi = pl.program_id(0)
    pj = pl.program_id(1)
    ii = pi * _BR + jax.lax.broadcasted_iota(jnp.int32, (_BR, _BR), 0)
    jj = pj * _BR + jax.lax.broadcasted_iota(jnp.int32, (_BR, _BR), 1)
    indsi = inds_c_ref[pi]
    mi = mask_c_ref[pi]
    indsj = inds_r_ref[pj]
    mj = mask_r_ref[pj]
    hit = (ii == jj)
    hit = hit | ((indsi == jj) & (mi > 0))
    hit = hit | ((indsj == ii) & (mj > 0))
    out_ref[...] = hit.astype(jnp.float32)


def kernel(input):
    x = input.reshape(_L, _L)

    inds_c, mask_c = pl.pallas_call(
        _rowstat_body,
        grid=(_NB,),
        in_specs=[pl.BlockSpec((_BR, _L), lambda i: (i, 0))],
        out_specs=[
            pl.BlockSpec((1, _BR, 1), lambda i: (i, 0, 0)),
            pl.BlockSpec((1, _BR, 1), lambda i: (i, 0, 0)),
        ],
        out_shape=[
            jax.ShapeDtypeStruct((_NB, _BR, 1), jnp.int32),
            jax.ShapeDtypeStruct((_NB, _BR, 1), jnp.int32),
        ],
    )(x)

    inds_r = inds_c.reshape(_NB, 1, _BR)
    mask_r = mask_c.reshape(_NB, 1, _BR)

    full = lambda i, j: (0, 0, 0)
    out2d = pl.pallas_call(
        _assemble_body,
        grid=(_NB, _NB),
        in_specs=[
            pl.BlockSpec((_NB, _BR, 1), full),
            pl.BlockSpec((_NB, 1, _BR), full),
            pl.BlockSpec((_NB, _BR, 1), full),
            pl.BlockSpec((_NB, 1, _BR), full),
        ],
        out_specs=pl.BlockSpec((_BR, _BR), lambda i, j: (i, j)),
        out_shape=jax.ShapeDtypeStruct((_L, _L), jnp.float32),
    )(inds_c, inds_r, mask_c, mask_r)

    return out2d.reshape(input.shape)
